# baseline (speedup 1.0000x reference)
import jax
import jax.numpy as jnp
from jax.experimental import pallas as pl

B_USERS = 256
B_ITEMS = 256


def _body(*refs):
  o_ref = refs[-1]
  acc = jnp.zeros((B_USERS, B_ITEMS), jnp.float32)
  for r in refs[:-1]:
    acc += r[0, 0]
  o_ref[...] = acc


_call = pl.pallas_call(
    _body,
    out_shape=jax.ShapeDtypeStruct((B_USERS, B_ITEMS), jnp.float32),
)


@jax.jit
def kernel(user_ids, item_ids, user_table, item_table):
  chunks = ([user_table[i * 256:(i + 1) * 256] for i in range(4)]
            + [item_table[i * 256:(i + 1) * 256] for i in range(4)])
  return _call(*chunks)


# speedup vs baseline: 1.5535x; 1.5535x over previous
import jax
import jax.numpy as jnp
from jax import lax
from jax.experimental import pallas as pl
from jax.experimental.pallas import tpu as pltpu

B_USERS = 256
B_ITEMS = 256
HIDDEN_DIM = 128
N_ROWS = 1024
BLK = 512
K_STEPS = N_ROWS // BLK


def _body(uid_ref, iid_ref, utab_ref, itab_ref, o_ref, u_acc, v_acc):
  k = pl.program_id(0)
  uid = uid_ref[0]
  iid = iid_ref[0]
  rows = k * BLK + lax.broadcasted_iota(jnp.int32, (B_USERS, BLK), 1)
  pu = (uid[:, None] == rows).astype(jnp.float32)
  pv = (iid[:, None] == rows).astype(jnp.float32)
  du = jnp.dot(pu, utab_ref[...], preferred_element_type=jnp.float32)
  dv = jnp.dot(pv, itab_ref[...], preferred_element_type=jnp.float32)

  @pl.when(k == 0)
  def _():
    u_acc[...] = du
    v_acc[...] = dv

  @pl.when(k == K_STEPS - 1)
  def _():
    o_ref[...] = lax.dot_general(
        u_acc[...] + du, v_acc[...] + dv,
        dimension_numbers=(((1,), (1,)), ((), ())),
        preferred_element_type=jnp.float32)


_call = pl.pallas_call(
    _body,
    grid=(K_STEPS,),
    in_specs=[
        pl.BlockSpec((1, B_USERS), lambda k: (0, 0)),
        pl.BlockSpec((1, B_ITEMS), lambda k: (0, 0)),
        pl.BlockSpec((BLK, HIDDEN_DIM), lambda k: (k, 0)),
        pl.BlockSpec((BLK, HIDDEN_DIM), lambda k: (k, 0)),
    ],
    out_specs=pl.BlockSpec((B_USERS, B_ITEMS), lambda k: (0, 0)),
    out_shape=jax.ShapeDtypeStruct((B_USERS, B_ITEMS), jnp.float32),
    scratch_shapes=[
        pltpu.VMEM((B_USERS, HIDDEN_DIM), jnp.float32),
        pltpu.VMEM((B_ITEMS, HIDDEN_DIM), jnp.float32),
    ],
)


@jax.jit
def kernel(user_ids, item_ids, user_table, item_table):
  return _call(user_ids.reshape(1, B_USERS), item_ids.reshape(1, B_ITEMS),
               user_table, item_table)
